# Initial kernel scaffold; baseline (speedup 1.0000x reference)
import jax
import jax.numpy as jnp
from jax.experimental import pallas as pl


def kernel(prop, index):
    raise NotImplementedError("write your pallas kernel here")


# speedup vs baseline: 5.4389x; 5.4389x over previous
import functools

import jax
import jax.numpy as jnp
from jax import lax
from jax.experimental import pallas as pl
from jax.experimental.pallas import tpu as pltpu
from jax.experimental.pallas import tpu_sc as plsc

CHUNK = 128
NC, NS = 2, 16
SEG_A = 624
CPAD = 10240
CSL = CPAD // NS


def _sc_accumulate(p2d, idx3, zrows, zcnt, ones, num_segments):
    N, D = p2d.shape
    n_chunks = N // CHUNK
    NW = NC * NS
    base_n = n_chunks // NW
    rem = n_chunks - base_n * NW
    tail = SEG_A - (SEG_A // CHUNK) * CHUNK
    seg_rem0 = num_segments - NS * SEG_A

    mesh = plsc.VectorSubcoreMesh(core_axis_name="c", subcore_axis_name="s")

    @functools.partial(
        pl.kernel,
        out_type=(
            jax.ShapeDtypeStruct((NC, num_segments, D), jnp.float32),
            jax.ShapeDtypeStruct((NC, CPAD), jnp.float32),
        ),
        mesh=mesh,
        scratch_types=[
            pltpu.VMEM((CHUNK, D), jnp.float32),
            pltpu.VMEM((1, CHUNK), jnp.int32),
            pltpu.VMEM((CHUNK,), jnp.float32),
            pltpu.VMEM((CSL,), jnp.float32),
            pltpu.VMEM_SHARED((num_segments, D), jnp.float32),
            pltpu.VMEM_SHARED((CPAD,), jnp.float32),
        ],
    )
    def k(p_hbm, i_hbm, zr_hbm, zc_hbm, on_hbm, sums_out, cnts_out,
          rows_v, idx_v, ones_v, cnt_v, sums_sh, cnts_sh):
        c = lax.axis_index("c")
        s = lax.axis_index("s")
        w = c * NS + s

        pltpu.sync_copy(zr_hbm, rows_v)
        pltpu.sync_copy(on_hbm, ones_v)

        row0 = s * SEG_A
        for j in range(SEG_A // CHUNK):
            pltpu.sync_copy(rows_v, sums_sh.at[pl.ds(row0 + j * CHUNK, CHUNK)])
        pltpu.sync_copy(rows_v.at[pl.ds(0, tail)],
                        sums_sh.at[pl.ds(row0 + (SEG_A // CHUNK) * CHUNK, tail)])

        @pl.when(s == 0)
        def _zero_rem():
            pltpu.sync_copy(rows_v.at[pl.ds(0, seg_rem0)],
                            sums_sh.at[pl.ds(NS * SEG_A, seg_rem0)])

        pltpu.sync_copy(zc_hbm, cnt_v)
        pltpu.sync_copy(cnt_v, cnts_sh.at[pl.ds(s * CSL, CSL)])

        plsc.subcore_barrier()

        start = w * base_n + jnp.minimum(w, rem)
        n_w = base_n + jnp.where(w < rem, 1, 0)

        def body(i, carry):
            ch = start + i
            pltpu.sync_copy(p_hbm.at[pl.ds(ch * CHUNK, CHUNK)], rows_v)
            pltpu.sync_copy(i_hbm.at[ch], idx_v)
            pltpu.sync_copy(rows_v, sums_sh.at[idx_v.at[0]], add=True)
            pltpu.sync_copy(ones_v, cnts_sh.at[idx_v.at[0]], add=True)
            return carry

        lax.fori_loop(0, n_w, body, 0)

        plsc.subcore_barrier()

        for j in range(SEG_A // CHUNK + 1):
            nrows = CHUNK if j < SEG_A // CHUNK else tail
            pltpu.sync_copy(sums_sh.at[pl.ds(row0 + j * CHUNK, nrows)],
                            rows_v.at[pl.ds(0, nrows)])
            pltpu.sync_copy(rows_v.at[pl.ds(0, nrows)],
                            sums_out.at[c, pl.ds(row0 + j * CHUNK, nrows)])

        @pl.when(s == 0)
        def _write_rem():
            pltpu.sync_copy(sums_sh.at[pl.ds(NS * SEG_A, seg_rem0)],
                            rows_v.at[pl.ds(0, seg_rem0)])
            pltpu.sync_copy(rows_v.at[pl.ds(0, seg_rem0)],
                            sums_out.at[c, pl.ds(NS * SEG_A, seg_rem0)])

        pltpu.sync_copy(cnts_sh.at[pl.ds(s * CSL, CSL)], cnt_v)
        pltpu.sync_copy(cnt_v, cnts_out.at[c, pl.ds(s * CSL, CSL)])

    return k(p2d, idx3, zrows, zcnt, ones)


def _combine(sums, cnts):
    _, S, D = sums.shape
    BS = 1000

    def body(s_ref, c_ref, o_ref):
        tot = s_ref[0] + s_ref[1]
        cnt = c_ref[0, :, 0] + c_ref[1, :, 0]
        o_ref[...] = tot / jnp.maximum(cnt, 1.0)[:, None]

    return pl.pallas_call(
        body,
        grid=(S // BS,),
        in_specs=[
            pl.BlockSpec((2, BS, D), lambda i: (0, i, 0)),
            pl.BlockSpec((2, BS, 1), lambda i: (0, i, 0)),
        ],
        out_specs=pl.BlockSpec((BS, D), lambda i: (i, 0)),
        out_shape=jax.ShapeDtypeStruct((S, D), jnp.float32),
    )(sums, cnts)


def kernel(prop, index):
    B, N, D = prop.shape
    num_segments = 10000
    p2d = prop.reshape(N, D)
    idx3 = index.reshape(N // CHUNK, 1, CHUNK)
    zrows = jnp.zeros((CHUNK, D), jnp.float32)
    zcnt = jnp.zeros((CSL,), jnp.float32)
    ones = jnp.ones((CHUNK,), jnp.float32)
    sums, cnts = _sc_accumulate(p2d, idx3, zrows, zcnt, ones, num_segments)
    out = _combine(sums, cnts[:, :num_segments].reshape(NC, num_segments, 1))
    return out.reshape(B, num_segments, D)
